# baseline (speedup 1.0000x reference)
import functools

import jax
import jax.numpy as jnp
from jax import lax
from jax.experimental import pallas as pl
from jax.experimental.pallas import tpu as pltpu
from jax.experimental.pallas import tpu_sc as plsc

BATCH = 16384
EMB = 64
NC = 2
NS = 16
NW = NC * NS
BPW = BATCH // NW
NCHUNK = 4
CHUNK = BPW // NCHUNK
LANES = 16


def _sc_body(users_hbm, pos_hbm, neg_hbm, item_hbm, table_hbm,
             pos_out_hbm, neg_out_hbm,
             uidx, pidx, nidx, urows, prows, nrows, outp, outn, sem):
    wid = lax.axis_index("s") * NC + lax.axis_index("c")
    base = wid * BPW

    for j in range(NCHUNK):
        src = pl.ds(base + j * CHUNK, CHUNK)
        pltpu.sync_copy(users_hbm.at[src], uidx.at[j])
        pltpu.sync_copy(pos_hbm.at[src], pidx.at[j])
        pltpu.sync_copy(neg_hbm.at[src], nidx.at[j])

    copies = []
    for j in range(NCHUNK):
        dst = pl.ds(j * CHUNK, CHUNK)
        copies.append(pltpu.async_copy(table_hbm.at[uidx.at[j]], urows.at[dst], sem))
        copies.append(pltpu.async_copy(item_hbm.at[pidx.at[j]], prows.at[dst], sem))
        copies.append(pltpu.async_copy(item_hbm.at[nidx.at[j]], nrows.at[dst], sem))
    for c in copies:
        c.wait()

    lanes = lax.iota(jnp.int32, LANES)

    perms = {h: lanes ^ h for h in (1, 2, 4, 8)}
    masks = {h: (lanes & h) != 0 for h in (1, 2, 4, 8)}

    dnums = lax.GatherDimensionNumbers(
        offset_dims=(), collapsed_slice_dims=(0,), start_index_map=(0,))

    def permute(v, idx):
        return lax.gather(v, idx[:, None], dnums, (1,),
                          mode=lax.GatherScatterMode.PROMISE_IN_BOUNDS)

    def combine(a, b, h):
        pa = permute(a, perms[h])
        pb = permute(b, perms[h])
        return jnp.where(masks[h], pb, a) + jnp.where(masks[h], b, pa)

    def tree(vs):
        h = 1
        while len(vs) > 1:
            vs = [combine(vs[i], vs[i + 1], h) for i in range(0, len(vs), 2)]
            h *= 2
        return vs[0]

    def group(g, carry):
        sp, sn = [], []
        for j in range(LANES):
            row = g * LANES + j
            pp = jnp.zeros((LANES,), jnp.float32)
            nn = jnp.zeros((LANES,), jnp.float32)
            for k in range(EMB // LANES):
                sl = pl.ds(k * LANES, LANES)
                u = urows[row, sl]
                pp = pp + u * prows[row, sl]
                nn = nn + u * nrows[row, sl]
            sp.append(pp)
            sn.append(nn)
        outp[pl.ds(g * LANES, LANES)] = tree(sp)
        outn[pl.ds(g * LANES, LANES)] = tree(sn)
        return carry

    lax.fori_loop(0, BPW // LANES, group, 0)

    pltpu.sync_copy(outp, pos_out_hbm.at[pl.ds(base, BPW)])
    pltpu.sync_copy(outn, neg_out_hbm.at[pl.ds(base, BPW)])


@functools.partial(
    pl.kernel,
    out_type=(
        jax.ShapeDtypeStruct((BATCH,), jnp.float32),
        jax.ShapeDtypeStruct((BATCH,), jnp.float32),
    ),
    mesh=plsc.VectorSubcoreMesh(core_axis_name="c", subcore_axis_name="s"),
    compiler_params=pltpu.CompilerParams(use_tc_tiling_on_sc=False),
    scratch_types=[
        pltpu.VMEM((NCHUNK, CHUNK), jnp.int32),
        pltpu.VMEM((NCHUNK, CHUNK), jnp.int32),
        pltpu.VMEM((NCHUNK, CHUNK), jnp.int32),
        pltpu.VMEM((BPW, EMB), jnp.float32),
        pltpu.VMEM((BPW, EMB), jnp.float32),
        pltpu.VMEM((BPW, EMB), jnp.float32),
        pltpu.VMEM((BPW,), jnp.float32),
        pltpu.VMEM((BPW,), jnp.float32),
        pltpu.SemaphoreType.DMA,
    ],
)
def _scores_sc(users_hbm, pos_hbm, neg_hbm, item_hbm, table_hbm,
               pos_out_hbm, neg_out_hbm, *scratch):
    _sc_body(users_hbm, pos_hbm, neg_hbm, item_hbm, table_hbm,
             pos_out_hbm, neg_out_hbm, *scratch)


@jax.jit
def kernel(users, pos_items, neg_items, all_item_embs, user_table):
    users = users.astype(jnp.int32)
    pos_items = pos_items.astype(jnp.int32)
    neg_items = neg_items.astype(jnp.int32)
    pos_scores, neg_scores = _scores_sc(
        users, pos_items, neg_items, all_item_embs, user_table)
    return (pos_scores, neg_scores)
